# baseline (speedup 1.0000x reference)
import functools

import jax
import jax.numpy as jnp
from jax import lax
from jax.experimental import pallas as pl
from jax.experimental.pallas import tpu as pltpu


def _body(logits_ref, hyp_ref, out_ref, *, nt):
    t = pl.program_id(1)
    h = hyp_ref[0, 0]
    tb = h.shape[0]
    v = logits_ref.shape[2]
    vc = 2048
    g = jnp.zeros((tb, 1), jnp.float32)
    s = jnp.zeros((tb, 1), jnp.float32)
    col = lax.broadcasted_iota(jnp.int32, (tb, vc), 1)
    for c in range(v // vc):
        xc = logits_ref[0, :, pl.ds(c * vc, vc)]
        g = g + jnp.sum(jnp.where(col == (h - c * vc), xc, 0.0), axis=1, keepdims=True)
        s = s + jnp.sum(jnp.exp(xc), axis=1, keepdims=True)
    partial = jnp.sum(g - jnp.log(s)).reshape(1, 1)

    @pl.when(t == 0)
    def _():
        out_ref[0] = jnp.zeros((1, 1), jnp.float32)

    out_ref[0] += partial


def kernel(logits, hyp):
    b, t, v = logits.shape
    tb = 256
    nt = t // tb
    hyp4 = hyp.astype(jnp.int32).reshape(b, nt, tb, 1)

    out = pl.pallas_call(
        functools.partial(_body, nt=nt),
        grid=(b, nt),
        in_specs=[
            pl.BlockSpec((1, tb, v), lambda i, j: (i, j, 0)),
            pl.BlockSpec((1, 1, tb, 1), lambda i, j: (i, j, 0, 0)),
        ],
        out_specs=pl.BlockSpec((1, 1, 1), lambda i, j: (i, 0, 0)),
        out_shape=jax.ShapeDtypeStruct((b, 1, 1), jnp.float32),
        compiler_params=pltpu.CompilerParams(
            dimension_semantics=("arbitrary", "arbitrary"),
        ),
    )(logits, hyp4)
    return out[:, 0, 0]
